# baseline (speedup 1.0000x reference)
import jax
import jax.numpy as jnp
from jax.experimental import pallas as pl
from jax.experimental.pallas import tpu as pltpu

P = 10000
E = 128
H = 128
B, S = 64, 200
M = S * B
TAB_ROWS = P + 8
EIDX_LEN = (S + 4) * B
GU = 32
CH = 512


def _body(e_idx, w_idx, wih_ev, wih_od, whh_t, b2, e2_hbm, wa_hbm,
          out_ref, e2_tab, wa_tab, xbuf_a, xbuf_b, hs_sc, wbuf_a, wbuf_b,
          sems):
    cp0 = pltpu.make_async_copy(e2_hbm, e2_tab, sems.at[0])
    cp1 = pltpu.make_async_copy(wa_hbm, wa_tab, sems.at[1])
    cp0.start()
    cp1.start()
    cp0.wait()

    def gather_x(dst, s):
        base = s * B
        for i in range(B):
            idx = e_idx[base + i]
            cb = pl.multiple_of((idx >> 3) << 3, 8)
            dst[pl.ds(i, 1), :] = pltpu.roll(
                e2_tab[pl.ds(cb, 8), :], -(idx & 7), axis=0)[0:1, :]

    gather_x(xbuf_a, 0)
    gather_x(xbuf_b, 1)

    def lstm_step(s, xbuf, h, c):
        u = xbuf[...]
        x_lo = pltpu.bitcast(jax.lax.shift_left(u, 16), jnp.float32)
        x_hi = pltpu.bitcast(jnp.bitwise_and(u, jnp.int32(-65536)), jnp.float32)
        g = (jnp.dot(x_lo, wih_ev[...], preferred_element_type=jnp.float32)
             + jnp.dot(x_hi, wih_od[...], preferred_element_type=jnp.float32)
             + jnp.dot(h, whh_t[...], preferred_element_type=jnp.float32)
             + b2[...])
        gi = jax.nn.sigmoid(g[:, 0:H])
        gf = jax.nn.sigmoid(g[:, H:2 * H])
        gg = jnp.tanh(g[:, 2 * H:3 * H])
        go = jax.nn.sigmoid(g[:, 3 * H:4 * H])
        c = gf * c + gi * gg
        h = go * jnp.tanh(c)
        hs_sc[pl.ds(pl.multiple_of(s * B, B), B), :] = h
        gather_x(xbuf, s + 2)
        return h, c

    def step2(t, carry):
        h, c = carry
        s0 = t * 2
        h, c = lstm_step(s0, xbuf_a, h, c)
        h, c = lstm_step(s0 + 1, xbuf_b, h, c)
        return (h, c)

    h0 = jnp.zeros((B, H), jnp.float32)
    jax.lax.fori_loop(0, S // 2, step2, (h0, h0))

    cp1.wait()
    for k in range(M // CH):
        cb0 = k * CH
        wbuf = wbuf_a if (k % 2 == 0) else wbuf_b

        def wgather(t, _, wbuf=wbuf, cb0=cb0):
            basej = t * GU
            for i in range(GU):
                j = basej + i
                wi = w_idx[cb0 + j]
                wb = pl.multiple_of((wi >> 3) << 3, 8)
                chunk = wa_tab[pl.ds(wb, 8), :]
                wbuf[pl.ds(j, 1), :] = pltpu.roll(chunk, -(wi & 7), axis=0)[0:1, :]
            return 0
        jax.lax.fori_loop(0, CH // GU, wgather, 0)

        hc = hs_sc[cb0:cb0 + CH, :]
        u = wbuf[...]
        w_lo = pltpu.bitcast(jax.lax.shift_left(u, 16), jnp.float32)
        b_hi = pltpu.bitcast(jnp.bitwise_and(u, jnp.int32(-65536)), jnp.float32)
        r = jnp.sum(hc * w_lo + b_hi, axis=1, keepdims=True)
        out_ref[cb0:cb0 + CH, :] = jax.nn.sigmoid(r)


def kernel(X, y, pro_embed, W_ih, W_hh, b_ih, b_hh, W_out, b_out):
    f32 = jnp.float32
    X = X.astype(jnp.int32)
    y = y.astype(jnp.int32)

    p_bits = jax.lax.bitcast_convert_type(
        pro_embed.astype(jnp.bfloat16), jnp.uint16)
    pair = (jnp.left_shift(p_bits[:, 1::2].astype(jnp.uint32), 16)
            | p_bits[:, 0::2].astype(jnp.uint32))
    zh = jnp.zeros((P, E // 2), jnp.uint32)
    e2_pack = jax.lax.bitcast_convert_type(jnp.concatenate([
        jnp.concatenate([pair, zh], axis=1),
        jnp.concatenate([zh, pair], axis=1),
        jnp.zeros((2 * TAB_ROWS - 2 * P, E), jnp.uint32),
    ], axis=0), jnp.int32)

    w_full = jnp.concatenate(
        [W_out, jnp.zeros((TAB_ROWS - P, H), f32)], axis=0)
    b_full = jnp.concatenate(
        [b_out, jnp.zeros((TAB_ROWS - P,), f32)]).at[P].set(-1e30)
    w_bits = jax.lax.bitcast_convert_type(
        w_full.astype(jnp.bfloat16), jnp.uint16).astype(jnp.uint32)
    b_bits = jax.lax.bitcast_convert_type(
        b_full.astype(jnp.bfloat16), jnp.uint16).astype(jnp.uint32)
    hi = jnp.concatenate(
        [b_bits[:, None], jnp.zeros((TAB_ROWS, H - 1), jnp.uint32)], axis=1)
    wa_pack = jax.lax.bitcast_convert_type(
        jnp.left_shift(hi, 16) | w_bits, jnp.int32)

    pad = EIDX_LEN - M
    yt = y.T
    e_idx = jnp.concatenate(
        [jnp.where(yt == -1, 2 * P, X.T + yt * P).reshape(M),
         jnp.zeros((pad,), jnp.int32)])
    Xn = jnp.concatenate([X[:, 1:], jnp.zeros((B, 1), jnp.int32)], axis=1)
    w_idx = jnp.where(Xn.T == 0, P, Xn.T - 1).reshape(M)

    wih_t = W_ih.T
    wih_ev = wih_t[0::2, :]
    wih_od = wih_t[1::2, :]
    whh_t = W_hh.T
    b2 = (b_ih + b_hh).reshape(1, 4 * H)

    out = pl.pallas_call(
        _body,
        in_specs=[
            pl.BlockSpec(memory_space=pltpu.SMEM),
            pl.BlockSpec(memory_space=pltpu.SMEM),
            pl.BlockSpec(memory_space=pltpu.VMEM),
            pl.BlockSpec(memory_space=pltpu.VMEM),
            pl.BlockSpec(memory_space=pltpu.VMEM),
            pl.BlockSpec(memory_space=pltpu.VMEM),
            pl.BlockSpec(memory_space=pl.ANY),
            pl.BlockSpec(memory_space=pl.ANY),
        ],
        out_specs=pl.BlockSpec(memory_space=pltpu.VMEM),
        out_shape=jax.ShapeDtypeStruct((M, 1), f32),
        scratch_shapes=[
            pltpu.VMEM((2 * TAB_ROWS, E), jnp.int32),
            pltpu.VMEM((TAB_ROWS, H), jnp.int32),
            pltpu.VMEM((B, E), jnp.int32),
            pltpu.VMEM((B, E), jnp.int32),
            pltpu.VMEM((M, H), f32),
            pltpu.VMEM((CH, H), jnp.int32),
            pltpu.VMEM((CH, H), jnp.int32),
            pltpu.SemaphoreType.DMA((2,)),
        ],
        compiler_params=pltpu.CompilerParams(
            vmem_limit_bytes=48 * 1024 * 1024,
        ),
        name="dkt_pebg_fused",
    )(e_idx, w_idx, wih_ev, wih_od, whh_t, b2, e2_pack, wa_pack)

    return out.reshape(S, B)[:S - 1].T


# speedup vs baseline: 1.0051x; 1.0051x over previous
import jax
import jax.numpy as jnp
from jax.experimental import pallas as pl
from jax.experimental.pallas import tpu as pltpu

P = 10000
E = 128
H = 128
B, S = 64, 200
M = S * B
TAB_ROWS = P + 8
EIDX_LEN = (S + 4) * B
GU = 32
CH = 512


def _body(e_idx, w_idx, wih_ev, whh_t, b2, e2_hbm, wa_hbm,
          out_ref, e2_tab, wa_tab, xbuf_a, xbuf_b, hs_sc, wbuf_a, wbuf_b,
          sems):
    cp0 = pltpu.make_async_copy(e2_hbm, e2_tab, sems.at[0])
    cp1 = pltpu.make_async_copy(wa_hbm, wa_tab, sems.at[1])
    cp0.start()
    cp1.start()
    cp0.wait()

    def gather_x(dst, s):
        base = s * B
        for i in range(B):
            idx = e_idx[base + i]
            cb = pl.multiple_of((idx >> 3) << 3, 8)
            row = pltpu.roll(
                e2_tab[pl.ds(cb, 8), :], -(idx & 7), axis=0)[0:1, :]
            dst[pl.ds(i, 1), 0:E] = pltpu.bitcast(
                jax.lax.shift_left(row, 16), jnp.float32)
            dst[pl.ds(i, 1), E:2 * E] = pltpu.bitcast(
                jnp.bitwise_and(row, jnp.int32(-65536)), jnp.float32)

    gather_x(xbuf_a, 0)
    gather_x(xbuf_b, 1)

    def lstm_step(s, xbuf, h, c):
        g = (jnp.dot(xbuf[...], wih_ev[...], preferred_element_type=jnp.float32)
             + jnp.dot(h, whh_t[...], preferred_element_type=jnp.float32)
             + b2[...])
        gi = jax.nn.sigmoid(g[:, 0:H])
        gf = jax.nn.sigmoid(g[:, H:2 * H])
        gg = jnp.tanh(g[:, 2 * H:3 * H])
        go = jax.nn.sigmoid(g[:, 3 * H:4 * H])
        c = gf * c + gi * gg
        h = go * jnp.tanh(c)
        hs_sc[pl.ds(pl.multiple_of(s * B, B), B), :] = h
        gather_x(xbuf, s + 2)
        return h, c

    def step2(t, carry):
        h, c = carry
        s0 = t * 2
        h, c = lstm_step(s0, xbuf_a, h, c)
        h, c = lstm_step(s0 + 1, xbuf_b, h, c)
        return (h, c)

    h0 = jnp.zeros((B, H), jnp.float32)
    jax.lax.fori_loop(0, S // 2, step2, (h0, h0))

    cp1.wait()
    for k in range(M // CH):
        cb0 = k * CH
        wbuf = wbuf_a if (k % 2 == 0) else wbuf_b

        def wgather(t, _, wbuf=wbuf, cb0=cb0):
            basej = t * GU
            for i in range(GU):
                j = basej + i
                wi = w_idx[cb0 + j]
                wb = pl.multiple_of((wi >> 3) << 3, 8)
                chunk = wa_tab[pl.ds(wb, 8), :]
                wbuf[pl.ds(j, 1), :] = pltpu.roll(chunk, -(wi & 7), axis=0)[0:1, :]
            return 0
        jax.lax.fori_loop(0, CH // GU, wgather, 0)

        hc = hs_sc[cb0:cb0 + CH, :]
        u = wbuf[...]
        w_lo = pltpu.bitcast(jax.lax.shift_left(u, 16), jnp.float32)
        b_hi = pltpu.bitcast(jnp.bitwise_and(u, jnp.int32(-65536)), jnp.float32)
        r = jnp.sum(hc * w_lo + b_hi, axis=1, keepdims=True)
        out_ref[cb0:cb0 + CH, :] = jax.nn.sigmoid(r)


def kernel(X, y, pro_embed, W_ih, W_hh, b_ih, b_hh, W_out, b_out):
    f32 = jnp.float32
    X = X.astype(jnp.int32)
    y = y.astype(jnp.int32)

    p_bits = jax.lax.bitcast_convert_type(
        pro_embed.astype(jnp.bfloat16), jnp.uint16)
    pair = (jnp.left_shift(p_bits[:, 1::2].astype(jnp.uint32), 16)
            | p_bits[:, 0::2].astype(jnp.uint32))
    zh = jnp.zeros((P, E // 2), jnp.uint32)
    e2_pack = jax.lax.bitcast_convert_type(jnp.concatenate([
        jnp.concatenate([pair, zh], axis=1),
        jnp.concatenate([zh, pair], axis=1),
        jnp.zeros((2 * TAB_ROWS - 2 * P, E), jnp.uint32),
    ], axis=0), jnp.int32)

    w_full = jnp.concatenate(
        [W_out, jnp.zeros((TAB_ROWS - P, H), f32)], axis=0)
    b_full = jnp.concatenate(
        [b_out, jnp.zeros((TAB_ROWS - P,), f32)]).at[P].set(-1e30)
    w_bits = jax.lax.bitcast_convert_type(
        w_full.astype(jnp.bfloat16), jnp.uint16).astype(jnp.uint32)
    b_bits = jax.lax.bitcast_convert_type(
        b_full.astype(jnp.bfloat16), jnp.uint16).astype(jnp.uint32)
    hi = jnp.concatenate(
        [b_bits[:, None], jnp.zeros((TAB_ROWS, H - 1), jnp.uint32)], axis=1)
    wa_pack = jax.lax.bitcast_convert_type(
        jnp.left_shift(hi, 16) | w_bits, jnp.int32)

    pad = EIDX_LEN - M
    yt = y.T
    e_idx = jnp.concatenate(
        [jnp.where(yt == -1, 2 * P, X.T + yt * P).reshape(M),
         jnp.zeros((pad,), jnp.int32)])
    Xn = jnp.concatenate([X[:, 1:], jnp.zeros((B, 1), jnp.int32)], axis=1)
    w_idx = jnp.where(Xn.T == 0, P, Xn.T - 1).reshape(M)

    wih_t = W_ih.T
    wih_ev = jnp.concatenate([wih_t[0::2, :], wih_t[1::2, :]], axis=0)
    whh_t = W_hh.T
    b2 = (b_ih + b_hh).reshape(1, 4 * H)

    out = pl.pallas_call(
        _body,
        in_specs=[
            pl.BlockSpec(memory_space=pltpu.SMEM),
            pl.BlockSpec(memory_space=pltpu.SMEM),
            pl.BlockSpec(memory_space=pltpu.VMEM),
            pl.BlockSpec(memory_space=pltpu.VMEM),
            pl.BlockSpec(memory_space=pltpu.VMEM),
            pl.BlockSpec(memory_space=pl.ANY),
            pl.BlockSpec(memory_space=pl.ANY),
        ],
        out_specs=pl.BlockSpec(memory_space=pltpu.VMEM),
        out_shape=jax.ShapeDtypeStruct((M, 1), f32),
        scratch_shapes=[
            pltpu.VMEM((2 * TAB_ROWS, E), jnp.int32),
            pltpu.VMEM((TAB_ROWS, H), jnp.int32),
            pltpu.VMEM((B, 2 * E), f32),
            pltpu.VMEM((B, 2 * E), f32),
            pltpu.VMEM((M, H), f32),
            pltpu.VMEM((CH, H), jnp.int32),
            pltpu.VMEM((CH, H), jnp.int32),
            pltpu.SemaphoreType.DMA((2,)),
        ],
        compiler_params=pltpu.CompilerParams(
            vmem_limit_bytes=48 * 1024 * 1024,
        ),
        name="dkt_pebg_fused",
    )(e_idx, w_idx, wih_ev, whh_t, b2, e2_pack, wa_pack)

    return out.reshape(S, B)[:S - 1].T


# speedup vs baseline: 2.5464x; 2.5335x over previous
import jax
import jax.numpy as jnp
from jax.experimental import pallas as pl
from jax.experimental.pallas import tpu as pltpu

P = 10000
E = 128
H = 128
B, S = 64, 200
M = S * B
TAB_ROWS = P + 8
EIDX_LEN = (S + 4) * B
GU = 32
CH = 512


def _body(e_idx, w_idx, wih_ev, whh_t, b2, e2_hbm, wa_hbm,
          out_ref, e2_tab, wa_tab, xbuf_a, xbuf_b, hs_sc, wbuf_a, wbuf_b,
          sems):
    cp0 = pltpu.make_async_copy(e2_hbm, e2_tab, sems.at[0])
    cp1 = pltpu.make_async_copy(wa_hbm, wa_tab, sems.at[1])
    cp0.start()
    cp1.start()
    cp0.wait()

    def gather_x(dst, s):
        base = s * B
        for i in range(B):
            idx = e_idx[base + i]
            cb = pl.multiple_of((idx >> 3) << 3, 8)
            row = pltpu.roll(
                e2_tab[pl.ds(cb, 8), :], -(idx & 7), axis=0)[0:1, :]
            dst[pl.ds(i, 1), 0:E] = pltpu.bitcast(
                jax.lax.shift_left(row, 16), jnp.float32)
            dst[pl.ds(i, 1), E:2 * E] = pltpu.bitcast(
                jnp.bitwise_and(row, jnp.int32(-65536)), jnp.float32)

    gather_x(xbuf_a, 0)
    gather_x(xbuf_b, 1)

    def lstm_step(s, xbuf, h, c):
        g = (jnp.dot(xbuf[...], wih_ev[...], preferred_element_type=jnp.float32)
             + jnp.dot(h, whh_t[...], preferred_element_type=jnp.float32)
             + b2[...])
        gi = jax.nn.sigmoid(g[:, 0:H])
        gf = jax.nn.sigmoid(g[:, H:2 * H])
        gg = jnp.tanh(g[:, 2 * H:3 * H])
        go = jax.nn.sigmoid(g[:, 3 * H:4 * H])
        c = gf * c + gi * gg
        h = go * jnp.tanh(c)
        hs_sc[pl.ds(pl.multiple_of(s * B, B), B), :] = h
        gather_x(xbuf, s + 2)
        return h, c

    def step2(t, carry):
        h, c = carry
        s0 = t * 2
        h, c = lstm_step(s0, xbuf_a, h, c)
        h, c = lstm_step(s0 + 1, xbuf_b, h, c)
        return (h, c)

    h0 = jnp.zeros((B, H), jnp.float32)
    jax.lax.fori_loop(0, S // 2, step2, (h0, h0))

    cp1.wait()
    for k in range(M // CH):
        cb0 = k * CH
        wbuf = wbuf_a if (k % 2 == 0) else wbuf_b

        def wgather(t, _, wbuf=wbuf, cb0=cb0):
            basej = t * GU
            for i in range(GU):
                j = basej + i
                wi = w_idx[cb0 + j]
                wb = pl.multiple_of((wi >> 3) << 3, 8)
                chunk = wa_tab[pl.ds(wb, 8), :]
                wbuf[pl.ds(j, 1), :] = pltpu.roll(chunk, -(wi & 7), axis=0)[0:1, :]
            return 0
        jax.lax.fori_loop(0, CH // GU, wgather, 0)

        hc = hs_sc[cb0:cb0 + CH, :]
        u = wbuf[...]
        w_lo = pltpu.bitcast(jax.lax.shift_left(u, 16), jnp.float32)
        b_hi = pltpu.bitcast(jnp.bitwise_and(u, jnp.int32(-65536)), jnp.float32)
        r = jnp.sum(hc * w_lo + b_hi, axis=1, keepdims=True)
        out_ref[cb0:cb0 + CH, :] = jax.nn.sigmoid(r)


def kernel(X, y, pro_embed, W_ih, W_hh, b_ih, b_hh, W_out, b_out):
    f32 = jnp.float32
    X = X.astype(jnp.int32)
    y = y.astype(jnp.int32)

    hi_bits = jax.lax.bitcast_convert_type(
        pro_embed.astype(jnp.bfloat16).astype(f32), jnp.uint32)
    e2_pack = jax.lax.bitcast_convert_type(jnp.concatenate([
        jnp.right_shift(hi_bits, 16),
        hi_bits,
        jnp.zeros((2 * TAB_ROWS - 2 * P, E), jnp.uint32),
    ], axis=0), jnp.int32)

    w_full = jnp.concatenate(
        [W_out, jnp.zeros((TAB_ROWS - P, H), f32)], axis=0)
    b_full = jnp.concatenate(
        [b_out, jnp.zeros((TAB_ROWS - P,), f32)]).at[P].set(-1e30)
    w_bits = jnp.right_shift(jax.lax.bitcast_convert_type(
        w_full.astype(jnp.bfloat16).astype(f32), jnp.uint32), 16)
    b_hi_bits = jax.lax.bitcast_convert_type(
        b_full.astype(jnp.bfloat16).astype(f32), jnp.uint32)
    hi = jnp.concatenate(
        [b_hi_bits[:, None], jnp.zeros((TAB_ROWS, H - 1), jnp.uint32)], axis=1)
    wa_pack = jax.lax.bitcast_convert_type(hi | w_bits, jnp.int32)

    pad = EIDX_LEN - M
    yt = y.T
    e_idx = jnp.concatenate(
        [jnp.where(yt == -1, 2 * P, X.T + yt * P).reshape(M),
         jnp.zeros((pad,), jnp.int32)])
    Xn = jnp.concatenate([X[:, 1:], jnp.zeros((B, 1), jnp.int32)], axis=1)
    w_idx = jnp.where(Xn.T == 0, P, Xn.T - 1).reshape(M)

    wih_ev = W_ih.T
    whh_t = W_hh.T
    b2 = (b_ih + b_hh).reshape(1, 4 * H)

    out = pl.pallas_call(
        _body,
        in_specs=[
            pl.BlockSpec(memory_space=pltpu.SMEM),
            pl.BlockSpec(memory_space=pltpu.SMEM),
            pl.BlockSpec(memory_space=pltpu.VMEM),
            pl.BlockSpec(memory_space=pltpu.VMEM),
            pl.BlockSpec(memory_space=pltpu.VMEM),
            pl.BlockSpec(memory_space=pl.ANY),
            pl.BlockSpec(memory_space=pl.ANY),
        ],
        out_specs=pl.BlockSpec(memory_space=pltpu.VMEM),
        out_shape=jax.ShapeDtypeStruct((M, 1), f32),
        scratch_shapes=[
            pltpu.VMEM((2 * TAB_ROWS, E), jnp.int32),
            pltpu.VMEM((TAB_ROWS, H), jnp.int32),
            pltpu.VMEM((B, 2 * E), f32),
            pltpu.VMEM((B, 2 * E), f32),
            pltpu.VMEM((M, H), f32),
            pltpu.VMEM((CH, H), jnp.int32),
            pltpu.VMEM((CH, H), jnp.int32),
            pltpu.SemaphoreType.DMA((2,)),
        ],
        compiler_params=pltpu.CompilerParams(
            vmem_limit_bytes=48 * 1024 * 1024,
        ),
        name="dkt_pebg_fused",
    )(e_idx, w_idx, wih_ev, whh_t, b2, e2_pack, wa_pack)

    return out.reshape(S, B)[:S - 1].T


# speedup vs baseline: 2.7971x; 1.0985x over previous
import jax
import jax.numpy as jnp
from jax.experimental import pallas as pl
from jax.experimental.pallas import tpu as pltpu

P = 10000
E = 128
H = 128
B, S = 64, 200
M = S * B
TAB_ROWS = P + 8
EIDX_LEN = (S + 4) * B
GU = 32
CH = 512


def _body(e_idx, w_idx, wih_ev, whh_t, b2, e2_hbm, wa_hbm,
          out_ref, e2_tab, wa_tab, xbuf_a, xbuf_b, hs_sc, wt_sc,
          sems):
    cp0 = pltpu.make_async_copy(e2_hbm, e2_tab, sems.at[0])
    cp1 = pltpu.make_async_copy(wa_hbm, wa_tab, sems.at[1])
    cp0.start()
    cp1.start()
    cp0.wait()

    def gather_x(dst, s):
        base = s * B
        for i in range(B):
            idx = e_idx[base + i]
            cb = pl.multiple_of((idx >> 3) << 3, 8)
            row = pltpu.roll(
                e2_tab[pl.ds(cb, 8), :], -(idx & 7), axis=0)[0:1, :]
            dst[pl.ds(i, 1), 0:E] = pltpu.bitcast(
                jax.lax.shift_left(row, 16), jnp.float32)
            dst[pl.ds(i, 1), E:2 * E] = pltpu.bitcast(
                jnp.bitwise_and(row, jnp.int32(-65536)), jnp.float32)

    gather_x(xbuf_a, 0)
    gather_x(xbuf_b, 1)
    cp1.wait()

    def lstm_step(s, xbuf, h, c):
        g = (jnp.dot(xbuf[...], wih_ev[...], preferred_element_type=jnp.float32)
             + jnp.dot(h, whh_t[...], preferred_element_type=jnp.float32)
             + b2[...])
        gi = jax.nn.sigmoid(g[:, 0:H])
        gf = jax.nn.sigmoid(g[:, H:2 * H])
        gg = jnp.tanh(g[:, 2 * H:3 * H])
        go = jax.nn.sigmoid(g[:, 3 * H:4 * H])
        c = gf * c + gi * gg
        h = go * jnp.tanh(c)
        hs_sc[pl.ds(pl.multiple_of(s * B, B), B), :] = h
        gather_x(xbuf, s + 2)
        return h, c

    def step2(t, carry):
        h, c = carry
        s0 = t * 2
        h, c = lstm_step(s0, xbuf_a, h, c)
        h, c = lstm_step(s0 + 1, xbuf_b, h, c)
        wbase = t * 2 * B
        for i in range(2 * B):
            wi = w_idx[wbase + i]
            wb = pl.multiple_of((wi >> 3) << 3, 8)
            wt_sc[pl.ds(wbase + i, 1), :] = pltpu.roll(
                wa_tab[pl.ds(wb, 8), :], -(wi & 7), axis=0)[0:1, :]
        return (h, c)

    h0 = jnp.zeros((B, H), jnp.float32)
    jax.lax.fori_loop(0, S // 2, step2, (h0, h0))

    for k in range(M // CH):
        cb0 = k * CH
        hc = hs_sc[cb0:cb0 + CH, :]
        u = wt_sc[cb0:cb0 + CH, :]
        w_lo = pltpu.bitcast(jax.lax.shift_left(u, 16), jnp.float32)
        b_hi = pltpu.bitcast(jnp.bitwise_and(u, jnp.int32(-65536)), jnp.float32)
        r = jnp.sum(hc * w_lo + b_hi, axis=1, keepdims=True)
        out_ref[cb0:cb0 + CH, :] = jax.nn.sigmoid(r)


def kernel(X, y, pro_embed, W_ih, W_hh, b_ih, b_hh, W_out, b_out):
    f32 = jnp.float32
    X = X.astype(jnp.int32)
    y = y.astype(jnp.int32)

    hi_bits = jax.lax.bitcast_convert_type(
        pro_embed.astype(jnp.bfloat16).astype(f32), jnp.uint32)
    e2_pack = jax.lax.bitcast_convert_type(jnp.concatenate([
        jnp.right_shift(hi_bits, 16),
        hi_bits,
        jnp.zeros((2 * TAB_ROWS - 2 * P, E), jnp.uint32),
    ], axis=0), jnp.int32)

    w_full = jnp.concatenate(
        [W_out, jnp.zeros((TAB_ROWS - P, H), f32)], axis=0)
    b_full = jnp.concatenate(
        [b_out, jnp.zeros((TAB_ROWS - P,), f32)]).at[P].set(-1e30)
    w_bits = jnp.right_shift(jax.lax.bitcast_convert_type(
        w_full.astype(jnp.bfloat16).astype(f32), jnp.uint32), 16)
    b_hi_bits = jax.lax.bitcast_convert_type(
        b_full.astype(jnp.bfloat16).astype(f32), jnp.uint32)
    hi = jnp.concatenate(
        [b_hi_bits[:, None], jnp.zeros((TAB_ROWS, H - 1), jnp.uint32)], axis=1)
    wa_pack = jax.lax.bitcast_convert_type(hi | w_bits, jnp.int32)

    pad = EIDX_LEN - M
    yt = y.T
    e_idx = jnp.concatenate(
        [jnp.where(yt == -1, 2 * P, X.T + yt * P).reshape(M),
         jnp.zeros((pad,), jnp.int32)])
    Xn = jnp.concatenate([X[:, 1:], jnp.zeros((B, 1), jnp.int32)], axis=1)
    w_idx = jnp.where(Xn.T == 0, P, Xn.T - 1).reshape(M)

    wih_ev = W_ih.T
    whh_t = W_hh.T
    b2 = (b_ih + b_hh).reshape(1, 4 * H)

    out = pl.pallas_call(
        _body,
        in_specs=[
            pl.BlockSpec(memory_space=pltpu.SMEM),
            pl.BlockSpec(memory_space=pltpu.SMEM),
            pl.BlockSpec(memory_space=pltpu.VMEM),
            pl.BlockSpec(memory_space=pltpu.VMEM),
            pl.BlockSpec(memory_space=pltpu.VMEM),
            pl.BlockSpec(memory_space=pl.ANY),
            pl.BlockSpec(memory_space=pl.ANY),
        ],
        out_specs=pl.BlockSpec(memory_space=pltpu.VMEM),
        out_shape=jax.ShapeDtypeStruct((M, 1), f32),
        scratch_shapes=[
            pltpu.VMEM((2 * TAB_ROWS, E), jnp.int32),
            pltpu.VMEM((TAB_ROWS, H), jnp.int32),
            pltpu.VMEM((B, 2 * E), f32),
            pltpu.VMEM((B, 2 * E), f32),
            pltpu.VMEM((M, H), f32),
            pltpu.VMEM((M, H), jnp.int32),
            pltpu.SemaphoreType.DMA((2,)),
        ],
        compiler_params=pltpu.CompilerParams(
            vmem_limit_bytes=48 * 1024 * 1024,
        ),
        name="dkt_pebg_fused",
    )(e_idx, w_idx, wih_ev, whh_t, b2, e2_pack, wa_pack)

    return out.reshape(S, B)[:S - 1].T
